# baseline (speedup 1.0000x reference)
import functools

import jax
import jax.numpy as jnp
from jax import lax
from jax.experimental import pallas as pl
from jax.experimental.pallas import tpu as pltpu
from jax.experimental.pallas import tpu_sc as plsc

Q = 8
D = 768
R = 4
KCOEF = 0.1
BK = 2000
_BIG_I32 = 2147483647


def _score_topk_body(q_ref, x_ref, ml_ref, lf_ref, vals_ref, idx_ref):
    i = pl.program_id(0)

    @pl.when(i == 0)
    def _init():
        vals_ref[...] = jnp.full((Q, R), -jnp.inf, jnp.float32)
        idx_ref[...] = jnp.zeros((Q, R), jnp.int32)

    q = q_ref[...]
    x = x_ref[...]
    ml = ml_ref[...].reshape(1, BK).astype(jnp.float32)
    lf = lf_ref[...]

    qss = jnp.sum(q * q, axis=1, keepdims=True)
    qn = q * lax.rsqrt(jnp.maximum(qss, 1e-16))

    raw = lax.dot_general(
        qn, x, (((1,), (1,)), ((), ())),
        preferred_element_type=jnp.float32,
        precision=lax.Precision.HIGHEST,
    )
    xss = lax.dot_general(
        jnp.ones((1, D), jnp.float32), x * x, (((1,), (1,)), ((), ())),
        preferred_element_type=jnp.float32,
        precision=lax.Precision.HIGHEST,
    )
    semantic = raw * lax.rsqrt(jnp.maximum(xss, 1e-16))

    rel = jnp.abs(ml - lf) / jnp.maximum(ml, lf)
    combined = semantic * jnp.exp(-rel * KCOEF)

    giota = lax.broadcasted_iota(jnp.int32, (Q, BK), 1) + i * BK
    s = combined
    bvals, bidx = [], []
    for _ in range(R):
        m = jnp.max(s, axis=1, keepdims=True)
        am = jnp.min(jnp.where(s == m, giota, _BIG_I32), axis=1,
                     keepdims=True)
        bvals.append(m)
        bidx.append(am)
        s = jnp.where(giota == am, -jnp.inf, s)

    cat_v = jnp.concatenate([vals_ref[...]] + bvals, axis=1)
    cat_i = jnp.concatenate([idx_ref[...]] + bidx, axis=1)
    pos = lax.broadcasted_iota(jnp.int32, (Q, 2 * R), 1)
    nv, ni = [], []
    for _ in range(R):
        m = jnp.max(cat_v, axis=1, keepdims=True)
        p = jnp.min(jnp.where(cat_v == m, pos, _BIG_I32), axis=1,
                    keepdims=True)
        sel = pos == p
        nv.append(m)
        ni.append(jnp.sum(jnp.where(sel, cat_i, 0), axis=1, keepdims=True))
        cat_v = jnp.where(sel, -jnp.inf, cat_v)
    vals_ref[...] = jnp.concatenate(nv, axis=1)
    idx_ref[...] = jnp.concatenate(ni, axis=1)


def _score_topk(query, x, ml3, lf, interpret=False):
    nb = x.shape[0] // BK
    return pl.pallas_call(
        _score_topk_body,
        grid=(nb,),
        in_specs=[
            pl.BlockSpec((Q, D), lambda i: (0, 0)),
            pl.BlockSpec((BK, D), lambda i: (i, 0)),
            pl.BlockSpec((1, 1, BK), lambda i: (i, 0, 0)),
            pl.BlockSpec((Q, 1), lambda i: (0, 0)),
        ],
        out_specs=[
            pl.BlockSpec((Q, R), lambda i: (0, 0)),
            pl.BlockSpec((Q, R), lambda i: (0, 0)),
        ],
        out_shape=[
            jax.ShapeDtypeStruct((Q, R), jnp.float32),
            jax.ShapeDtypeStruct((Q, R), jnp.int32),
        ],
        interpret=interpret,
    )(query, x, ml3, lf)


@functools.lru_cache(maxsize=1)
def _build_gather_weight():
    mesh = plsc.VectorSubcoreMesh(core_axis_name="c", subcore_axis_name="s")

    @functools.partial(
        pl.kernel,
        mesh=mesh,
        out_type=jax.ShapeDtypeStruct((Q * R, D), jnp.float32),
        scratch_types=[
            pltpu.VMEM((Q * R,), jnp.int32),
            pltpu.VMEM((16,), jnp.float32),
            pltpu.VMEM((Q * R, D), jnp.float32),
            pltpu.VMEM((D,), jnp.float32),
            pltpu.SemaphoreType.DMA,
        ],
    )
    def gather_weight(idx_hbm, scb_hbm, table_hbm, out_hbm,
                      idx_v, scb_v, rows_v, out_v, sem):
        w = lax.axis_index("s") * 2 + lax.axis_index("c")
        pltpu.sync_copy(idx_hbm, idx_v)
        pltpu.async_copy(table_hbm.at[idx_v], rows_v, sem).wait()
        pltpu.sync_copy(scb_hbm.at[w], scb_v)
        score = scb_v[...]
        for j in range(D // 16):
            out_v[pl.ds(j * 16, 16)] = rows_v[w, pl.ds(j * 16, 16)] * score
        pltpu.sync_copy(out_v, out_hbm.at[w])

    return gather_weight


def kernel(query, text_features, lengths, motion_lengths):
    k = text_features.shape[0]
    nb = k // BK
    lf = lengths.astype(jnp.float32).reshape(Q, 1)
    ml3 = motion_lengths.astype(jnp.int32).reshape(nb, 1, BK)
    top_vals, top_idx = _score_topk(query, text_features, ml3, lf)
    score_bcast = jnp.broadcast_to(top_vals.reshape(Q * R, 1), (Q * R, 16))
    weighted = _build_gather_weight()(top_idx.reshape(Q * R),
                                      score_bcast,
                                      text_features)
    return weighted.reshape(Q, R, D), top_idx


# speedup vs baseline: 2.7375x; 2.7375x over previous
import functools

import jax
import jax.numpy as jnp
from jax import lax
from jax.experimental import pallas as pl
from jax.experimental.pallas import tpu as pltpu
from jax.experimental.pallas import tpu_sc as plsc

Q = 8
D = 768
R = 4
KCOEF = 0.1
BK = 2000
_BIG_I32 = 2147483647


def _score_topk_body(q_ref, x_ref, ml_ref, lf_ref, vals_ref, idx_ref):
    i = pl.program_id(0)

    @pl.when(i == 0)
    def _init():
        vals_ref[...] = jnp.full((R, Q), -jnp.inf, jnp.float32)
        idx_ref[...] = jnp.zeros((R, Q), jnp.int32)

    q = q_ref[...]
    x = x_ref[...]
    ml = ml_ref[...]
    lf = lf_ref[...]

    qss = jnp.sum(q * q, axis=1, keepdims=True)
    qn = q * lax.rsqrt(jnp.maximum(qss, 1e-16))

    rawT = lax.dot_general(
        x, qn, (((1,), (1,)), ((), ())),
        preferred_element_type=jnp.float32,
    )
    xss = jnp.sum(x * x, axis=1, keepdims=True)
    sem = rawT * lax.rsqrt(jnp.maximum(xss, 1e-16))

    rel = jnp.abs(ml - lf) / jnp.maximum(ml, lf)
    s = sem * jnp.exp(-rel * KCOEF)

    giota = lax.broadcasted_iota(jnp.int32, (BK, Q), 0) + i * BK
    bvals, bidx = [], []
    for _ in range(R):
        m = jnp.max(s, axis=0, keepdims=True)
        am = jnp.min(jnp.where(s == m, giota, _BIG_I32), axis=0,
                     keepdims=True)
        bvals.append(m)
        bidx.append(am)
        s = jnp.where(giota == am, -jnp.inf, s)

    cat_v = jnp.concatenate([vals_ref[...]] + bvals, axis=0)
    cat_i = jnp.concatenate([idx_ref[...]] + bidx, axis=0)
    pos = lax.broadcasted_iota(jnp.int32, (2 * R, Q), 0)
    nv, ni = [], []
    for _ in range(R):
        m = jnp.max(cat_v, axis=0, keepdims=True)
        p = jnp.min(jnp.where(cat_v == m, pos, _BIG_I32), axis=0,
                    keepdims=True)
        sel = pos == p
        nv.append(m)
        ni.append(jnp.sum(jnp.where(sel, cat_i, 0), axis=0, keepdims=True))
        cat_v = jnp.where(sel, -jnp.inf, cat_v)
    vals_ref[...] = jnp.concatenate(nv, axis=0)
    idx_ref[...] = jnp.concatenate(ni, axis=0)


def _score_topk(query, x, ml2, lf, interpret=False):
    nb = x.shape[0] // BK
    return pl.pallas_call(
        _score_topk_body,
        grid=(nb,),
        in_specs=[
            pl.BlockSpec((Q, D), lambda i: (0, 0)),
            pl.BlockSpec((BK, D), lambda i: (i, 0)),
            pl.BlockSpec((BK, 1), lambda i: (i, 0)),
            pl.BlockSpec((1, Q), lambda i: (0, 0)),
        ],
        out_specs=[
            pl.BlockSpec((R, Q), lambda i: (0, 0)),
            pl.BlockSpec((R, Q), lambda i: (0, 0)),
        ],
        out_shape=[
            jax.ShapeDtypeStruct((R, Q), jnp.float32),
            jax.ShapeDtypeStruct((R, Q), jnp.int32),
        ],
        interpret=interpret,
    )(query, x, ml2, lf)


@functools.lru_cache(maxsize=1)
def _build_gather_weight():
    mesh = plsc.VectorSubcoreMesh(core_axis_name="c", subcore_axis_name="s")

    @functools.partial(
        pl.kernel,
        mesh=mesh,
        out_type=jax.ShapeDtypeStruct((Q * R, D), jnp.float32),
        scratch_types=[
            pltpu.VMEM((Q * R,), jnp.int32),
            pltpu.VMEM((16,), jnp.float32),
            pltpu.VMEM((Q * R, D), jnp.float32),
            pltpu.VMEM((D,), jnp.float32),
            pltpu.SemaphoreType.DMA,
        ],
    )
    def gather_weight(idx_hbm, scb_hbm, table_hbm, out_hbm,
                      idx_v, scb_v, rows_v, out_v, sem):
        w = lax.axis_index("s") * 2 + lax.axis_index("c")
        pltpu.sync_copy(idx_hbm, idx_v)
        pltpu.async_copy(table_hbm.at[idx_v], rows_v, sem).wait()
        pltpu.sync_copy(scb_hbm.at[w], scb_v)
        score = scb_v[...]
        for j in range(D // 16):
            out_v[pl.ds(j * 16, 16)] = rows_v[w, pl.ds(j * 16, 16)] * score
        pltpu.sync_copy(out_v, out_hbm.at[w])

    return gather_weight


def kernel(query, text_features, lengths, motion_lengths):
    k = text_features.shape[0]
    lf = lengths.astype(jnp.float32).reshape(1, Q)
    ml2 = motion_lengths.astype(jnp.float32).reshape(k, 1)
    vals_t, idx_t = _score_topk(query, text_features, ml2, lf)
    top_vals = vals_t.T
    top_idx = idx_t.T
    score_bcast = jnp.broadcast_to(top_vals.reshape(Q * R, 1), (Q * R, 16))
    weighted = _build_gather_weight()(top_idx.reshape(Q * R),
                                      score_bcast,
                                      text_features)
    return weighted.reshape(Q, R, D), top_idx


# speedup vs baseline: 2.7966x; 1.0216x over previous
import functools

import jax
import jax.numpy as jnp
from jax import lax
from jax.experimental import pallas as pl
from jax.experimental.pallas import tpu as pltpu
from jax.experimental.pallas import tpu_sc as plsc

Q = 8
D = 768
R = 4
KCOEF = 0.1
BK = 4000
_BIG_I32 = 2147483647


def _score_topk_body(q_ref, x_ref, ml_ref, lf_ref, vals_ref, idx_ref):
    i = pl.program_id(0)

    @pl.when(i == 0)
    def _init():
        vals_ref[...] = jnp.full((R, Q), -jnp.inf, jnp.float32)
        idx_ref[...] = jnp.zeros((R, Q), jnp.int32)

    q = q_ref[...]
    x = x_ref[...]
    ml = ml_ref[...]
    lf = lf_ref[...]

    qss = jnp.sum(q * q, axis=1, keepdims=True)
    qn = q * lax.rsqrt(jnp.maximum(qss, 1e-16))

    rawT = lax.dot_general(
        x, qn, (((1,), (1,)), ((), ())),
        preferred_element_type=jnp.float32,
    )
    xss = jnp.sum(x * x, axis=1, keepdims=True)
    sem = rawT * lax.rsqrt(jnp.maximum(xss, 1e-16))

    rel = jnp.abs(ml - lf) / jnp.maximum(ml, lf)
    s = sem * jnp.exp(-rel * KCOEF)

    giota = lax.broadcasted_iota(jnp.int32, (BK, Q), 0) + i * BK
    bvals, bidx = [], []
    for _ in range(R):
        m = jnp.max(s, axis=0, keepdims=True)
        am = jnp.min(jnp.where(s == m, giota, _BIG_I32), axis=0,
                     keepdims=True)
        bvals.append(m)
        bidx.append(am)
        s = jnp.where(giota == am, -jnp.inf, s)

    cat_v = jnp.concatenate([vals_ref[...]] + bvals, axis=0)
    cat_i = jnp.concatenate([idx_ref[...]] + bidx, axis=0)
    pos = lax.broadcasted_iota(jnp.int32, (2 * R, Q), 0)
    nv, ni = [], []
    for _ in range(R):
        m = jnp.max(cat_v, axis=0, keepdims=True)
        p = jnp.min(jnp.where(cat_v == m, pos, _BIG_I32), axis=0,
                    keepdims=True)
        sel = pos == p
        nv.append(m)
        ni.append(jnp.sum(jnp.where(sel, cat_i, 0), axis=0, keepdims=True))
        cat_v = jnp.where(sel, -jnp.inf, cat_v)
    vals_ref[...] = jnp.concatenate(nv, axis=0)
    idx_ref[...] = jnp.concatenate(ni, axis=0)


def _score_topk(query, x, ml2, lf, interpret=False):
    nb = x.shape[0] // BK
    return pl.pallas_call(
        _score_topk_body,
        grid=(nb,),
        in_specs=[
            pl.BlockSpec((Q, D), lambda i: (0, 0)),
            pl.BlockSpec((BK, D), lambda i: (i, 0)),
            pl.BlockSpec((BK, 1), lambda i: (i, 0)),
            pl.BlockSpec((1, Q), lambda i: (0, 0)),
        ],
        out_specs=[
            pl.BlockSpec((R, Q), lambda i: (0, 0)),
            pl.BlockSpec((R, Q), lambda i: (0, 0)),
        ],
        out_shape=[
            jax.ShapeDtypeStruct((R, Q), jnp.float32),
            jax.ShapeDtypeStruct((R, Q), jnp.int32),
        ],
        interpret=interpret,
    )(query, x, ml2, lf)


@functools.lru_cache(maxsize=1)
def _build_gather_weight():
    mesh = plsc.VectorSubcoreMesh(core_axis_name="c", subcore_axis_name="s")

    @functools.partial(
        pl.kernel,
        mesh=mesh,
        out_type=jax.ShapeDtypeStruct((Q * R, D), jnp.float32),
        scratch_types=[
            pltpu.VMEM((Q * R,), jnp.int32),
            pltpu.VMEM((16,), jnp.float32),
            pltpu.VMEM((Q * R, D), jnp.float32),
            pltpu.VMEM((D,), jnp.float32),
            pltpu.SemaphoreType.DMA,
        ],
    )
    def gather_weight(idx_hbm, scb_hbm, table_hbm, out_hbm,
                      idx_v, scb_v, rows_v, out_v, sem):
        w = lax.axis_index("s") * 2 + lax.axis_index("c")
        pltpu.sync_copy(idx_hbm, idx_v)
        pltpu.async_copy(table_hbm.at[idx_v], rows_v, sem).wait()
        pltpu.sync_copy(scb_hbm.at[w], scb_v)
        score = scb_v[...]
        for j in range(D // 16):
            out_v[pl.ds(j * 16, 16)] = rows_v[w, pl.ds(j * 16, 16)] * score
        pltpu.sync_copy(out_v, out_hbm.at[w])

    return gather_weight


def kernel(query, text_features, lengths, motion_lengths):
    k = text_features.shape[0]
    lf = lengths.astype(jnp.float32).reshape(1, Q)
    ml2 = motion_lengths.astype(jnp.float32).reshape(k, 1)
    vals_t, idx_t = _score_topk(query, text_features, ml2, lf)
    top_vals = vals_t.T
    top_idx = idx_t.T
    score_bcast = jnp.broadcast_to(top_vals.reshape(Q * R, 1), (Q * R, 16))
    weighted = _build_gather_weight()(top_idx.reshape(Q * R),
                                      score_bcast,
                                      text_features)
    return weighted.reshape(Q, R, D), top_idx


# speedup vs baseline: 2.8911x; 1.0338x over previous
import functools

import jax
import jax.numpy as jnp
from jax import lax
from jax.experimental import pallas as pl
from jax.experimental.pallas import tpu as pltpu
from jax.experimental.pallas import tpu_sc as plsc

Q = 8
D = 768
R = 4
KCOEF = 0.1
BK = 2000
_BIG_I32 = 2147483647


def _block_top4(qn, x, ml, lf, base):
    rawT = lax.dot_general(
        x, qn, (((1,), (1,)), ((), ())),
        preferred_element_type=jnp.float32,
    )
    xss = jnp.sum(x * x, axis=1, keepdims=True)
    sem = rawT * lax.rsqrt(jnp.maximum(xss, 1e-16))

    rel = jnp.abs(ml - lf) / jnp.maximum(ml, lf)
    s = sem * jnp.exp(-rel * KCOEF)

    giota = lax.broadcasted_iota(jnp.int32, (BK, Q), 0) + base
    bvals, bidx = [], []
    for _ in range(R):
        m = jnp.max(s, axis=0, keepdims=True)
        am = jnp.min(jnp.where(s == m, giota, _BIG_I32), axis=0,
                     keepdims=True)
        bvals.append(m)
        bidx.append(am)
        s = jnp.where(giota == am, -jnp.inf, s)
    return bvals, bidx


def _score_topk_body(nbh, q_ref, xa_ref, xb_ref, mla_ref, mlb_ref, lf_ref,
                     vals_ref, idx_ref):
    i = pl.program_id(0)

    @pl.when(i == 0)
    def _init():
        vals_ref[...] = jnp.full((R, Q), -jnp.inf, jnp.float32)
        idx_ref[...] = jnp.zeros((R, Q), jnp.int32)

    q = q_ref[...]
    lf = lf_ref[...]

    qss = jnp.sum(q * q, axis=1, keepdims=True)
    qn = q * lax.rsqrt(jnp.maximum(qss, 1e-16))

    av, ai = _block_top4(qn, xa_ref[...], mla_ref[...], lf, i * BK)
    bv, bi = _block_top4(qn, xb_ref[...], mlb_ref[...], lf, (i + nbh) * BK)

    cat_v = jnp.concatenate([vals_ref[...]] + av + bv, axis=0)
    cat_i = jnp.concatenate([idx_ref[...]] + ai + bi, axis=0)
    pos = lax.broadcasted_iota(jnp.int32, (3 * R, Q), 0)
    nv, ni = [], []
    for _ in range(R):
        m = jnp.max(cat_v, axis=0, keepdims=True)
        p = jnp.min(jnp.where(cat_v == m, pos, _BIG_I32), axis=0,
                    keepdims=True)
        sel = pos == p
        nv.append(m)
        ni.append(jnp.sum(jnp.where(sel, cat_i, 0), axis=0, keepdims=True))
        cat_v = jnp.where(sel, -jnp.inf, cat_v)
    vals_ref[...] = jnp.concatenate(nv, axis=0)
    idx_ref[...] = jnp.concatenate(ni, axis=0)


def _score_topk(query, x, ml2, lf, interpret=False):
    nbh = x.shape[0] // (2 * BK)
    return pl.pallas_call(
        functools.partial(_score_topk_body, nbh),
        grid=(nbh,),
        in_specs=[
            pl.BlockSpec((Q, D), lambda i: (0, 0)),
            pl.BlockSpec((BK, D), lambda i: (i, 0)),
            pl.BlockSpec((BK, D), lambda i, nbh=nbh: (i + nbh, 0)),
            pl.BlockSpec((BK, 1), lambda i: (i, 0)),
            pl.BlockSpec((BK, 1), lambda i, nbh=nbh: (i + nbh, 0)),
            pl.BlockSpec((1, Q), lambda i: (0, 0)),
        ],
        out_specs=[
            pl.BlockSpec((R, Q), lambda i: (0, 0)),
            pl.BlockSpec((R, Q), lambda i: (0, 0)),
        ],
        out_shape=[
            jax.ShapeDtypeStruct((R, Q), jnp.float32),
            jax.ShapeDtypeStruct((R, Q), jnp.int32),
        ],
        interpret=interpret,
    )(query, x, x, ml2, ml2, lf)


@functools.lru_cache(maxsize=1)
def _build_gather_weight():
    mesh = plsc.VectorSubcoreMesh(core_axis_name="c", subcore_axis_name="s")

    @functools.partial(
        pl.kernel,
        mesh=mesh,
        out_type=jax.ShapeDtypeStruct((Q * R, D), jnp.float32),
        scratch_types=[
            pltpu.VMEM((Q * R,), jnp.int32),
            pltpu.VMEM((16,), jnp.float32),
            pltpu.VMEM((Q * R, D), jnp.float32),
            pltpu.VMEM((D,), jnp.float32),
            pltpu.SemaphoreType.DMA,
        ],
    )
    def gather_weight(idx_hbm, scb_hbm, table_hbm, out_hbm,
                      idx_v, scb_v, rows_v, out_v, sem):
        w = lax.axis_index("s") * 2 + lax.axis_index("c")
        pltpu.sync_copy(idx_hbm, idx_v)
        pltpu.async_copy(table_hbm.at[idx_v], rows_v, sem).wait()
        pltpu.sync_copy(scb_hbm.at[w], scb_v)
        score = scb_v[...]
        for j in range(D // 16):
            out_v[pl.ds(j * 16, 16)] = rows_v[w, pl.ds(j * 16, 16)] * score
        pltpu.sync_copy(out_v, out_hbm.at[w])

    return gather_weight


def kernel(query, text_features, lengths, motion_lengths):
    k = text_features.shape[0]
    lf = lengths.astype(jnp.float32).reshape(1, Q)
    ml2 = motion_lengths.astype(jnp.float32).reshape(k, 1)
    vals_t, idx_t = _score_topk(query, text_features, ml2, lf)
    top_vals = vals_t.T
    top_idx = idx_t.T
    score_bcast = jnp.broadcast_to(top_vals.reshape(Q * R, 1), (Q * R, 16))
    weighted = _build_gather_weight()(top_idx.reshape(Q * R),
                                      score_bcast,
                                      text_features)
    return weighted.reshape(Q, R, D), top_idx
